# baseline (speedup 1.0000x reference)
import functools

import jax
import jax.numpy as jnp
from jax import lax
from jax.experimental import pallas as pl
from jax.experimental.pallas import tpu as pltpu
from jax.experimental.pallas import tpu_sc as plsc

D_MODEL = 64
SCALE = 8.0
LANES = 16
NC, NS = 2, 16
NW = NC * NS
B = 4096 * 50
BPW = B // NW
CHUNK = 128
NCHUNK = BPW // CHUNK
NBUF = 5
NGROUP = NCHUNK // NBUF


def _scale_chunk(src, dst):
    def row(r, _):
        for j in range(D_MODEL // LANES):
            sl = pl.ds(j * LANES, LANES)
            dst[r, sl] = src[r, sl] * SCALE
        return 0

    lax.fori_loop(0, CHUNK, row, 0, unroll=2)


def _body(x_hbm, table_hbm, out_hbm, idx_v, *bufs):
    gb = bufs[0:NBUF]
    ob = bufs[NBUF:2 * NBUF]
    gs = bufs[2 * NBUF:3 * NBUF]
    ss = bufs[3 * NBUF:4 * NBUF]

    wid = lax.axis_index("s") * NC + lax.axis_index("c")

    pltpu.sync_copy(x_hbm.at[wid], idx_v)

    for b in range(NBUF):
        pltpu.async_copy(table_hbm.at[idx_v.at[b]], gb[b], gs[b])

    for b in range(NBUF):
        pltpu.make_async_copy(table_hbm.at[idx_v.at[b]], gb[b], gs[b]).wait()
        _scale_chunk(gb[b], ob[b])
        pltpu.async_copy(ob[b], out_hbm.at[wid, b], ss[b])
        pltpu.async_copy(table_hbm.at[idx_v.at[b + NBUF]], gb[b], gs[b])

    def group(g, _):
        i0 = g * NBUF
        for b in range(NBUF):
            ci = i0 + b
            pltpu.make_async_copy(
                table_hbm.at[idx_v.at[ci]], gb[b], gs[b]).wait()
            pltpu.make_async_copy(ob[b], out_hbm.at[wid, ci], ss[b]).wait()
            _scale_chunk(gb[b], ob[b])
            pltpu.async_copy(ob[b], out_hbm.at[wid, ci], ss[b])
            pltpu.async_copy(table_hbm.at[idx_v.at[ci + NBUF]], gb[b], gs[b])
        return 0

    lax.fori_loop(1, NGROUP - 1, group, 0)

    for b in range(NBUF):
        ci = (NGROUP - 1) * NBUF + b
        pltpu.make_async_copy(table_hbm.at[idx_v.at[ci]], gb[b], gs[b]).wait()
        pltpu.make_async_copy(ob[b], out_hbm.at[wid, ci], ss[b]).wait()
        _scale_chunk(gb[b], ob[b])
        pltpu.async_copy(ob[b], out_hbm.at[wid, ci], ss[b])

    for b in range(NBUF):
        ci = (NGROUP - 1) * NBUF + b
        pltpu.make_async_copy(ob[b], out_hbm.at[wid, ci], ss[b]).wait()


@jax.jit
def _embed(x_flat, table):
    mesh = plsc.VectorSubcoreMesh(core_axis_name="c", subcore_axis_name="s")
    scratch = (
        [pltpu.VMEM((NCHUNK, CHUNK), jnp.int32)]
        + [pltpu.VMEM((CHUNK, D_MODEL), jnp.float32) for _ in range(2 * NBUF)]
        + [pltpu.SemaphoreType.DMA for _ in range(2 * NBUF)]
    )
    k = pl.kernel(
        _body,
        out_type=jax.ShapeDtypeStruct((NW, NCHUNK, CHUNK, D_MODEL),
                                      jnp.float32),
        mesh=mesh,
        scratch_types=scratch,
        compiler_params=pltpu.CompilerParams(use_tc_tiling_on_sc=False),
    )
    return k(x_flat.reshape(NW, NCHUNK, CHUNK), table)


def kernel(x, table):
    out = _embed(x.reshape(-1), table)
    return out.reshape(x.shape[0], x.shape[1], D_MODEL)
